# baseline (speedup 1.0000x reference)
import jax
import jax.numpy as jnp
from jax import lax
from jax.experimental import pallas as pl
from jax.experimental.pallas import tpu as pltpu
from jax.experimental.pallas import tpu_sc as plsc

_L = 8192
_D = 768
_NW = 32
_RPW = _L // _NW
_W = 32
_NCH = _RPW // _W
_LANES = 16
_NTB = 3
_NPB = 2


def _sc_body(tok_hbm, pos_hbm, ids_hbm, spec_hbm, o_hbm,
             idx_v, pidx_v, spec_v,
             tb0, tb1, tb2, pb0, pb1,
             ts0, ts1, ts2, ts3, ts4, ts5, ps0, ps1, ps2, ps3,
             ws0, ws1, ws2):
    tbufs = (tb0, tb1, tb2)
    pbufs = (pb0, pb1)
    tsems = ((ts0, ts1), (ts2, ts3), (ts4, ts5))
    psems = ((ps0, ps1), (ps2, ps3))
    wsems = (ws0, ws1, ws2)

    c_id = lax.axis_index("c")
    s_id = lax.axis_index("s")
    wid = s_id * 2 + c_id
    is_w0 = wid == 0
    base = wid * _RPW

    pltpu.sync_copy(ids_hbm.at[0, pl.ds(base, _RPW)], idx_v)
    pltpu.sync_copy(spec_hbm.at[0, 0], spec_v)

    lane = lax.iota(jnp.int32, _LANES)
    for ch in range(_NCH):
        for k in range(0, _W, _LANES):
            vals = lane + (base + ch * _W + k - 1)
            if ch == 0 and k == 0:
                vals = jnp.maximum(vals, 0)
            pidx_v[ch, pl.ds(k, _LANES)] = vals

    _H = _W // 2

    def start_tok(c):
        buf = tbufs[c % _NTB]
        s0, s1 = tsems[c % _NTB]
        a = pltpu.async_copy(tok_hbm.at[idx_v.at[pl.ds(c * _W, _H)]],
                             buf.at[pl.ds(0, _H)], s0)
        b = pltpu.async_copy(tok_hbm.at[idx_v.at[pl.ds(c * _W + _H, _H)]],
                             buf.at[pl.ds(_H, _H)], s1)
        return (a, b)

    def start_pos(c):
        buf = pbufs[c % _NPB]
        s0, s1 = psems[c % _NPB]
        a = pltpu.async_copy(pos_hbm.at[pidx_v.at[c, pl.ds(0, _H)]],
                             buf.at[pl.ds(0, _H)], s0)
        b = pltpu.async_copy(pos_hbm.at[pidx_v.at[c, pl.ds(_H, _H)]],
                             buf.at[pl.ds(_H, _H)], s1)
        return (a, b)

    tok_cp = {}
    pos_cp = {}
    writes = {}
    tok_cp[0] = start_tok(0)
    pos_cp[0] = start_pos(0)
    tok_cp[1] = start_tok(1)
    pos_cp[1] = start_pos(1)

    for c in range(_NCH):
        b = c % _NTB
        tok_v = tbufs[b]
        pos_v = pbufs[c % _NPB]
        def add_rows(lo, hi):
            @pl.loop(lo, hi, step=2)
            def _row(r):
                for dr in range(2):
                    for col in range(0, _D, _LANES):
                        plsc.addupdate(tok_v.at[r + dr, pl.ds(col, _LANES)],
                                       pos_v[r + dr, pl.ds(col, _LANES)])

        tok_cp[c][0].wait()
        pos_cp[c][0].wait()
        if c == 0:
            @pl.when(is_w0)
            def _w0():
                add_rows(2, _H)
                for col in range(0, _D, _LANES):
                    tok_v[0, pl.ds(col, _LANES)] = (
                        tok_v[1, pl.ds(col, _LANES)]
                        + pos_v[0, pl.ds(col, _LANES)])
                    tok_v[1, pl.ds(col, _LANES)] = spec_v[pl.ds(col, _LANES)]

            @pl.when(jnp.logical_not(is_w0))
            def _rest():
                add_rows(0, _H)
        else:
            add_rows(0, _H)

        tok_cp[c][1].wait()
        pos_cp[c][1].wait()
        add_rows(_H, _W)

        writes[c] = pltpu.async_copy(
            tok_v, o_hbm.at[0, pl.ds(base + c * _W, _W)], wsems[b])

        nxt = c + 2
        if nxt < _NCH:
            if nxt - _NTB >= 0:
                writes[nxt - _NTB].wait()
            tok_cp[nxt] = start_tok(nxt)
            pos_cp[nxt] = start_pos(nxt)

    for c in range(_NCH - _NTB, _NCH):
        writes[c].wait()


@jax.jit
def _embed(token_embedding, position_embedding, input_ids, special_tok):
    mesh = plsc.VectorSubcoreMesh(core_axis_name="c", subcore_axis_name="s")
    run = pl.kernel(
        _sc_body,
        out_type=jax.ShapeDtypeStruct((1, _L, _D), jnp.float32),
        mesh=mesh,
        scratch_types=(
            [
                pltpu.VMEM((_RPW,), jnp.int32),
                pltpu.VMEM((_NCH, _W), jnp.int32),
                pltpu.VMEM((_D,), jnp.float32),
            ]
            + [pltpu.VMEM((_W, _D), jnp.float32)] * (_NTB + _NPB)
            + [pltpu.SemaphoreType.DMA] * 13
        ),
    )
    return run(token_embedding, position_embedding, input_ids, special_tok)


def kernel(input_ids, token_embedding, position_embedding, special_token_embedding):
    return _embed(token_embedding, position_embedding, input_ids,
                  special_token_embedding)


# speedup vs baseline: 1.0376x; 1.0376x over previous
import jax
import jax.numpy as jnp
from jax import lax
from jax.experimental import pallas as pl
from jax.experimental.pallas import tpu as pltpu
from jax.experimental.pallas import tpu_sc as plsc

_L = 8192
_D = 768
_NW = 32
_RPW = _L // _NW
_W = 32
_NCH = _RPW // _W
_LANES = 16
_NTB = 3
_NPB = 2


def _sc_body(tok_hbm, pos_hbm, ids_hbm, spec_hbm, o_hbm,
             idx_v, pidx_v, spec_v,
             tb0, tb1, tb2, pb0, pb1,
             ts0, ts1, ts2, ps0, ps1, ws0, ws1, ws2):
    tbufs = (tb0, tb1, tb2)
    pbufs = (pb0, pb1)
    tsems = (ts0, ts1, ts2)
    psems = (ps0, ps1)
    wsems = (ws0, ws1, ws2)

    c_id = lax.axis_index("c")
    s_id = lax.axis_index("s")
    wid = s_id * 2 + c_id
    is_w0 = wid == 0
    base = wid * _RPW

    pltpu.sync_copy(ids_hbm.at[0, pl.ds(base, _RPW)], idx_v)
    pltpu.sync_copy(spec_hbm.at[0, 0], spec_v)

    lane = lax.iota(jnp.int32, _LANES)
    for ch in range(_NCH):
        for k in range(0, _W, _LANES):
            vals = lane + (base + ch * _W + k - 1)
            if ch == 0 and k == 0:
                vals = jnp.maximum(vals, 0)
            pidx_v[ch, pl.ds(k, _LANES)] = vals

    _H = _W // 2

    def start_tok(c):
        buf = tbufs[c % _NTB]
        sem = tsems[c % _NTB]
        a = pltpu.async_copy(tok_hbm.at[idx_v.at[pl.ds(c * _W, _H)]],
                             buf.at[pl.ds(0, _H)], sem)
        b = pltpu.async_copy(tok_hbm.at[idx_v.at[pl.ds(c * _W + _H, _H)]],
                             buf.at[pl.ds(_H, _H)], sem)
        return (a, b)

    def start_pos(c):
        buf = pbufs[c % _NPB]
        sem = psems[c % _NPB]
        a = pltpu.async_copy(pos_hbm.at[pidx_v.at[c, pl.ds(0, _H)]],
                             buf.at[pl.ds(0, _H)], sem)
        b = pltpu.async_copy(pos_hbm.at[pidx_v.at[c, pl.ds(_H, _H)]],
                             buf.at[pl.ds(_H, _H)], sem)
        return (a, b)

    tok_cp = {}
    pos_cp = {}
    writes = {}
    tok_cp[0] = start_tok(0)
    pos_cp[0] = start_pos(0)
    tok_cp[1] = start_tok(1)
    pos_cp[1] = start_pos(1)

    for c in range(_NCH):
        b = c % _NTB
        tok_v = tbufs[b]
        pos_v = pbufs[c % _NPB]
        tok_cp[c][0].wait()
        tok_cp[c][1].wait()
        pos_cp[c][0].wait()
        pos_cp[c][1].wait()

        def add_rows(lo):
            @pl.loop(lo, _W, step=2)
            def _row(r):
                for dr in range(2):
                    for col in range(0, _D, _LANES):
                        plsc.addupdate(tok_v.at[r + dr, pl.ds(col, _LANES)],
                                       pos_v[r + dr, pl.ds(col, _LANES)])

        if c == 0:
            @pl.when(is_w0)
            def _w0():
                add_rows(2)
                for col in range(0, _D, _LANES):
                    tok_v[0, pl.ds(col, _LANES)] = (
                        tok_v[1, pl.ds(col, _LANES)]
                        + pos_v[0, pl.ds(col, _LANES)])
                    tok_v[1, pl.ds(col, _LANES)] = spec_v[pl.ds(col, _LANES)]

            @pl.when(jnp.logical_not(is_w0))
            def _rest():
                add_rows(0)
        else:
            add_rows(0)

        writes[c] = pltpu.async_copy(
            tok_v, o_hbm.at[0, pl.ds(base + c * _W, _W)], wsems[b])

        nxt = c + 2
        if nxt < _NCH:
            if nxt - _NTB >= 0:
                writes[nxt - _NTB].wait()
            tok_cp[nxt] = start_tok(nxt)
            pos_cp[nxt] = start_pos(nxt)

    for c in range(_NCH - _NTB, _NCH):
        writes[c].wait()


@jax.jit
def _embed(token_embedding, position_embedding, input_ids, special_tok):
    mesh = plsc.VectorSubcoreMesh(core_axis_name="c", subcore_axis_name="s")
    run = pl.kernel(
        _sc_body,
        out_type=jax.ShapeDtypeStruct((1, _L, _D), jnp.float32),
        mesh=mesh,
        scratch_types=(
            [
                pltpu.VMEM((_RPW,), jnp.int32),
                pltpu.VMEM((_NCH, _W), jnp.int32),
                pltpu.VMEM((_D,), jnp.float32),
            ]
            + [pltpu.VMEM((_W, _D), jnp.float32)] * (_NTB + _NPB)
            + [pltpu.SemaphoreType.DMA] * 8
        ),
    )
    return run(token_embedding, position_embedding, input_ids, special_tok)


def kernel(input_ids, token_embedding, position_embedding, special_token_embedding):
    return _embed(token_embedding, position_embedding, input_ids,
                  special_token_embedding)
